# baseline (speedup 1.0000x reference)
import functools

import jax
import jax.numpy as jnp
from jax import lax
from jax.experimental import pallas as pl
from jax.experimental.pallas import tpu as pltpu
from jax.experimental.pallas import tpu_sc as plsc

N = 10000
NPAD = 10240
F = 128
E = 320000
NC = 2
NS = 16
NW = NC * NS
EPT = 10240
CHUNK = 64
CHUNKS = EPT // CHUNK
IB = 16
NG = CHUNKS // IB
NBUF = 5
LEAD = NBUF - 1
SLACK = NBUF - LEAD
EPAD = EPT * NW
ROWS_PER_TILE = NPAD // NS
ZB = ROWS_PER_TILE // CHUNK

_MESH = plsc.VectorSubcoreMesh(
    core_axis_name="c", subcore_axis_name="s", num_cores=NC, num_subcores=NS)


@functools.partial(
    pl.kernel,
    out_type=jax.ShapeDtypeStruct((NC, NPAD), jnp.float32),
    mesh=_MESH,
    scratch_types=[
        pltpu.VMEM((CHUNKS, CHUNK), jnp.int32),
        pltpu.VMEM((CHUNK,), jnp.float32),
        pltpu.VMEM((ROWS_PER_TILE,), jnp.float32),
        pltpu.VMEM_SHARED((NPAD,), jnp.float32),
        pltpu.SemaphoreType.DMA,
    ],
)
def _deg_kernel(dstb_hbm, deg_hbm, idx_v, ones_v, zv, acc, dsem):
    c = lax.axis_index("c")
    s = lax.axis_index("s")
    pltpu.sync_copy(dstb_hbm.at[c, s], idx_v)
    for k in range(CHUNK // 16):
        ones_v[pl.ds(k * 16, 16)] = jnp.full((16,), 1.0, jnp.float32)
    for k in range(ROWS_PER_TILE // 16):
        zv[pl.ds(k * 16, 16)] = jnp.zeros((16,), jnp.float32)
    pltpu.sync_copy(zv, acc.at[pl.ds(s * ROWS_PER_TILE, ROWS_PER_TILE)])
    plsc.subcore_barrier()

    def body(i, _):
        for t in range(8):
            pltpu.async_copy(ones_v, acc.at[idx_v.at[i * 8 + t]], dsem,
                             add=True)
        for t in range(8):
            pltpu.make_async_copy(
                ones_v, acc.at[idx_v.at[i * 8 + t]], dsem).wait()
        return _

    lax.fori_loop(0, CHUNKS // 8, body, None)
    plsc.subcore_barrier()
    pltpu.sync_copy(acc.at[pl.ds(s * ROWS_PER_TILE, ROWS_PER_TILE)],
                    deg_hbm.at[c, pl.ds(s * ROWS_PER_TILE, ROWS_PER_TILE)])


@functools.partial(
    pl.kernel,
    out_type=jax.ShapeDtypeStruct((NC, NPAD, F), jnp.float32),
    mesh=_MESH,
    scratch_types=[
        pltpu.VMEM((2, IB, CHUNK), jnp.int32),
        pltpu.VMEM((2, IB, CHUNK), jnp.int32),
        pltpu.VMEM((NBUF, CHUNK, F), jnp.float32),
        pltpu.VMEM_SHARED((NPAD, F), jnp.float32),
        [pltpu.SemaphoreType.DMA] * NBUF,
        [pltpu.SemaphoreType.DMA] * NBUF,
        pltpu.SemaphoreType.DMA,
    ],
)
def _prop_kernel(g_hbm, srcb_hbm, dstb_hbm, p_hbm,
                 src_v, dst_v, rows_v, acc, gsems, ssems, sem_i):
    c = lax.axis_index("c")
    s = lax.axis_index("s")

    def loc(j):
        return (j // IB) % 2, j % IB, j % NBUF

    def gather_start(j):
        slot, b, r = loc(j)
        pltpu.async_copy(g_hbm.at[src_v.at[slot, b]], rows_v.at[r], gsems[r])

    def gather_wait(j):
        slot, b, r = loc(j)
        pltpu.make_async_copy(
            g_hbm.at[src_v.at[slot, b]], rows_v.at[r], gsems[r]).wait()

    def scat_start(j):
        slot, b, r = loc(j)
        pltpu.async_copy(rows_v.at[r], acc.at[dst_v.at[slot, b]],
                         ssems[r], add=True)

    def scat_wait(j):
        slot, b, r = loc(j)
        pltpu.make_async_copy(
            rows_v.at[r], acc.at[dst_v.at[slot, b]], ssems[r]).wait()

    def idx_start(g, slot):
        pltpu.async_copy(srcb_hbm.at[c, s, pl.ds(g * IB, IB)],
                         src_v.at[slot], sem_i)
        pltpu.async_copy(dstb_hbm.at[c, s, pl.ds(g * IB, IB)],
                         dst_v.at[slot], sem_i)

    def idx_wait(g, slot):
        mk = pltpu.make_async_copy
        mk(srcb_hbm.at[c, s, pl.ds(g * IB, IB)], src_v.at[slot], sem_i).wait()
        mk(dstb_hbm.at[c, s, pl.ds(g * IB, IB)], dst_v.at[slot], sem_i).wait()

    pltpu.sync_copy(srcb_hbm.at[c, s, pl.ds(0, IB)], src_v.at[0])
    pltpu.sync_copy(dstb_hbm.at[c, s, pl.ds(0, IB)], dst_v.at[0])

    def zfill(i, _):
        for k in range(F // 16):
            rows_v[0, i, pl.ds(k * 16, 16)] = jnp.zeros((16,), jnp.float32)
        return _

    lax.fori_loop(0, CHUNK, zfill, None)
    for t in range(ZB):
        pltpu.sync_copy(
            rows_v.at[0],
            acc.at[pl.ds(s * ROWS_PER_TILE + t * CHUNK, CHUNK)])
    plsc.subcore_barrier()

    idx_start(1, 1)
    for j in range(LEAD):
        gather_start(j)

    for g in range(NG):
        slot = g % 2
        for b in range(IB):
            j = g * IB + b
            gather_wait(j)
            if b == IB - LEAD and g + 1 < NG:
                idx_wait(g + 1, 1 - slot)
            if j >= SLACK:
                scat_wait(j - SLACK)
            if b == 1 and 1 <= g < NG - 1:
                idx_start(g + 1, 1 - slot)
            if j + LEAD < CHUNKS:
                gather_start(j + LEAD)
            scat_start(j)
    for t in range(SLACK):
        scat_wait(CHUNKS - SLACK + t)
    plsc.subcore_barrier()
    pltpu.sync_copy(acc.at[pl.ds(s * ROWS_PER_TILE, ROWS_PER_TILE)],
                    p_hbm.at[c, pl.ds(s * ROWS_PER_TILE, ROWS_PER_TILE)])


_RB = 1024
_GRID = NPAD // _RB


def _row_block(i):
    return (i, 0)


def _mm1_body(x_ref, w_ref, h_ref):
    h_ref[...] = jnp.dot(x_ref[...], w_ref[...],
                         preferred_element_type=jnp.float32)


def _mm1(xpad, w1):
    return pl.pallas_call(
        _mm1_body,
        grid=(_GRID,),
        in_specs=[
            pl.BlockSpec((_RB, F), _row_block),
            pl.BlockSpec((F, F), lambda i: (0, 0)),
        ],
        out_specs=pl.BlockSpec((_RB, F), _row_block),
        out_shape=jax.ShapeDtypeStruct((NPAD, F), jnp.float32),
    )(xpad, w1)


def _stage1_body(d0_ref, d1_ref, h_ref, g_ref, dinv_ref):
    dinv = lax.rsqrt(1.0 + d0_ref[...] + d1_ref[...])
    g_ref[...] = dinv[:, None] * h_ref[...]
    dinv_ref[...] = dinv


def _stage1(d0, d1, h1):
    return pl.pallas_call(
        _stage1_body,
        grid=(_GRID,),
        in_specs=[
            pl.BlockSpec((_RB,), lambda i: (i,)),
            pl.BlockSpec((_RB,), lambda i: (i,)),
            pl.BlockSpec((_RB, F), _row_block),
        ],
        out_specs=[
            pl.BlockSpec((_RB, F), _row_block),
            pl.BlockSpec((_RB,), lambda i: (i,)),
        ],
        out_shape=[
            jax.ShapeDtypeStruct((NPAD, F), jnp.float32),
            jax.ShapeDtypeStruct((NPAD,), jnp.float32),
        ],
    )(d0, d1, h1)


def _stage_mid_body(p0_ref, p1_ref, g_ref, dinv_ref, b_ref, w_ref, gn_ref):
    dinv = dinv_ref[...]
    t = jnp.tanh(dinv[:, None] * (p0_ref[...] + p1_ref[...] + g_ref[...])
                 + b_ref[...])
    gn_ref[...] = dinv[:, None] * jnp.dot(
        t, w_ref[...], preferred_element_type=jnp.float32)


def _stage_mid(p0, p1, g, dinv, b_row, w_next):
    return pl.pallas_call(
        _stage_mid_body,
        grid=(_GRID,),
        in_specs=[
            pl.BlockSpec((_RB, F), _row_block),
            pl.BlockSpec((_RB, F), _row_block),
            pl.BlockSpec((_RB, F), _row_block),
            pl.BlockSpec((_RB,), lambda i: (i,)),
            pl.BlockSpec((1, F), lambda i: (0, 0)),
            pl.BlockSpec((F, F), lambda i: (0, 0)),
        ],
        out_specs=pl.BlockSpec((_RB, F), _row_block),
        out_shape=jax.ShapeDtypeStruct((NPAD, F), jnp.float32),
    )(p0, p1, g, dinv, b_row, w_next)


_RBF = 2000


def _stage_fin_body(p0_ref, p1_ref, g_ref, dinv_ref, b_ref, o_ref):
    dinv = dinv_ref[...]
    o_ref[...] = (dinv * (p0_ref[...] + p1_ref[...] + g_ref[...])
                  + b_ref[...])


def _stage_fin(p0, p1, g, dinv, b_row):
    return pl.pallas_call(
        _stage_fin_body,
        grid=(N // _RBF,),
        in_specs=[
            pl.BlockSpec((_RBF, F), _row_block),
            pl.BlockSpec((_RBF, F), _row_block),
            pl.BlockSpec((_RBF, F), _row_block),
            pl.BlockSpec((_RBF, 1), lambda i: (i, 0)),
            pl.BlockSpec((1, F), lambda i: (0, 0)),
        ],
        out_specs=pl.BlockSpec((_RBF, F), _row_block),
        out_shape=jax.ShapeDtypeStruct((N, F), jnp.float32),
    )(p0, p1, g, dinv.reshape(NPAD, 1), b_row)


def kernel(x, edge_index, W1, b1, W2, b2, W3, b3):
    src = edge_index[0].astype(jnp.int32)
    dst = edge_index[1].astype(jnp.int32)
    pad = N + (jnp.arange(EPAD - E, dtype=jnp.int32) % (NPAD - N))
    srcp = jnp.concatenate([src, pad]).reshape(NC, NS, CHUNKS, CHUNK)
    dstp = jnp.concatenate([dst, pad]).reshape(NC, NS, CHUNKS, CHUNK)
    xpad = jnp.pad(x, ((0, NPAD - N), (0, 0)))
    b1r, b2r, b3r = (b.reshape(1, F) for b in (b1, b2, b3))

    degp = _deg_kernel(dstp)
    h1 = _mm1(xpad, W1)
    g1, dinv = _stage1(degp[0], degp[1], h1)
    p = _prop_kernel(g1, srcp, dstp)
    g2 = _stage_mid(p[0], p[1], g1, dinv, b1r, W2)
    p = _prop_kernel(g2, srcp, dstp)
    g3 = _stage_mid(p[0], p[1], g2, dinv, b2r, W3)
    p = _prop_kernel(g3, srcp, dstp)
    return _stage_fin(p[0], p[1], g3, dinv, b3r)


# speedup vs baseline: 1.0186x; 1.0186x over previous
import functools

import jax
import jax.numpy as jnp
from jax import lax
from jax.experimental import pallas as pl
from jax.experimental.pallas import tpu as pltpu
from jax.experimental.pallas import tpu_sc as plsc

N = 10000
NPAD = 10240
F = 128
E = 320000
NC = 2
NS = 16
NW = NC * NS
EPT = 10240
CHUNK = 64
CHUNKS = EPT // CHUNK
IB = 16
NG = CHUNKS // IB
NBUF = 5
LEAD = NBUF - 1
SLACK = NBUF - LEAD
EPAD = EPT * NW
ROWS_PER_TILE = NPAD // NS
ZB = ROWS_PER_TILE // CHUNK

_MESH = plsc.VectorSubcoreMesh(
    core_axis_name="c", subcore_axis_name="s", num_cores=NC, num_subcores=NS)


@functools.partial(
    pl.kernel,
    out_type=jax.ShapeDtypeStruct((NC, NPAD), jnp.float32),
    mesh=_MESH,
    scratch_types=[
        pltpu.VMEM((CHUNKS, CHUNK), jnp.int32),
        pltpu.VMEM((CHUNK,), jnp.float32),
        pltpu.VMEM((ROWS_PER_TILE,), jnp.float32),
        pltpu.VMEM_SHARED((NPAD,), jnp.float32),
        pltpu.SemaphoreType.DMA,
    ],
)
def _deg_kernel(dstb_hbm, deg_hbm, idx_v, ones_v, zv, acc, dsem):
    c = lax.axis_index("c")
    s = lax.axis_index("s")
    pltpu.sync_copy(dstb_hbm.at[c, s], idx_v)
    for k in range(CHUNK // 16):
        ones_v[pl.ds(k * 16, 16)] = jnp.full((16,), 1.0, jnp.float32)
    for k in range(ROWS_PER_TILE // 16):
        zv[pl.ds(k * 16, 16)] = jnp.zeros((16,), jnp.float32)
    pltpu.sync_copy(zv, acc.at[pl.ds(s * ROWS_PER_TILE, ROWS_PER_TILE)])
    plsc.subcore_barrier()

    def body(i, _):
        for t in range(8):
            pltpu.async_copy(ones_v, acc.at[idx_v.at[i * 8 + t]], dsem,
                             add=True)
        for t in range(8):
            pltpu.make_async_copy(
                ones_v, acc.at[idx_v.at[i * 8 + t]], dsem).wait()
        return _

    lax.fori_loop(0, CHUNKS // 8, body, None)
    plsc.subcore_barrier()
    pltpu.sync_copy(acc.at[pl.ds(s * ROWS_PER_TILE, ROWS_PER_TILE)],
                    deg_hbm.at[c, pl.ds(s * ROWS_PER_TILE, ROWS_PER_TILE)])


@functools.partial(
    pl.kernel,
    out_type=jax.ShapeDtypeStruct((NC, NPAD, F), jnp.float32),
    mesh=_MESH,
    scratch_types=[
        pltpu.VMEM((2, IB, CHUNK), jnp.int32),
        pltpu.VMEM((2, IB, CHUNK), jnp.int32),
        pltpu.VMEM((NBUF, CHUNK, F), jnp.float32),
        pltpu.VMEM_SHARED((NPAD, F), jnp.float32),
        [pltpu.SemaphoreType.DMA] * NBUF,
        [pltpu.SemaphoreType.DMA] * NBUF,
        pltpu.SemaphoreType.DMA,
    ],
)
def _prop_kernel(g_hbm, srcb_hbm, dstb_hbm, p_hbm,
                 src_v, dst_v, rows_v, acc, gsems, ssems, sem_i):
    c = lax.axis_index("c")
    s = lax.axis_index("s")

    def loc(j):
        return (j // IB) % 2, j % IB, j % NBUF

    def gather_start(j):
        slot, b, r = loc(j)
        pltpu.async_copy(g_hbm.at[src_v.at[slot, b]], rows_v.at[r], gsems[r])

    def gather_wait(j):
        slot, b, r = loc(j)
        pltpu.make_async_copy(
            g_hbm.at[src_v.at[slot, b]], rows_v.at[r], gsems[r]).wait()

    def scat_start(j):
        slot, b, r = loc(j)
        pltpu.async_copy(rows_v.at[r], acc.at[dst_v.at[slot, b]],
                         ssems[r], add=True)

    def scat_wait(j):
        slot, b, r = loc(j)
        pltpu.make_async_copy(
            rows_v.at[r], acc.at[dst_v.at[slot, b]], ssems[r]).wait()

    def idx_start(g, slot):
        pltpu.async_copy(srcb_hbm.at[c, s, pl.ds(g * IB, IB)],
                         src_v.at[slot], sem_i)
        pltpu.async_copy(dstb_hbm.at[c, s, pl.ds(g * IB, IB)],
                         dst_v.at[slot], sem_i)

    def idx_wait(g, slot):
        mk = pltpu.make_async_copy
        mk(srcb_hbm.at[c, s, pl.ds(g * IB, IB)], src_v.at[slot], sem_i).wait()
        mk(dstb_hbm.at[c, s, pl.ds(g * IB, IB)], dst_v.at[slot], sem_i).wait()

    pltpu.sync_copy(srcb_hbm.at[c, s, pl.ds(0, IB)], src_v.at[0])
    pltpu.sync_copy(dstb_hbm.at[c, s, pl.ds(0, IB)], dst_v.at[0])

    def zfill(i, _):
        for k in range(F // 16):
            rows_v[0, i, pl.ds(k * 16, 16)] = jnp.zeros((16,), jnp.float32)
        return _

    lax.fori_loop(0, CHUNK, zfill, None)
    for t in range(ZB):
        pltpu.sync_copy(
            rows_v.at[0],
            acc.at[pl.ds(s * ROWS_PER_TILE + t * CHUNK, CHUNK)])
    plsc.subcore_barrier()

    idx_start(1, 1)
    for j in range(LEAD):
        gather_start(j)

    for g in range(NG):
        slot = g % 2
        for b in range(IB):
            j = g * IB + b
            gather_wait(j)
            if b == IB - LEAD and g + 1 < NG:
                idx_wait(g + 1, 1 - slot)
            if j >= SLACK:
                scat_wait(j - SLACK)
            if b == 1 and 1 <= g < NG - 1:
                idx_start(g + 1, 1 - slot)
            if j + LEAD < CHUNKS:
                gather_start(j + LEAD)
            scat_start(j)
    for t in range(SLACK):
        scat_wait(CHUNKS - SLACK + t)
    plsc.subcore_barrier()
    pltpu.sync_copy(acc.at[pl.ds(s * ROWS_PER_TILE, ROWS_PER_TILE)],
                    p_hbm.at[c, pl.ds(s * ROWS_PER_TILE, ROWS_PER_TILE)])


_RB = 1024
_GRID = NPAD // _RB


def _row_block(i):
    return (i, 0)


def _stage1_body(d0_ref, d1_ref, x_ref, w_ref, g_ref, dinv_ref):
    dinv = lax.rsqrt(1.0 + d0_ref[...] + d1_ref[...])
    h = jnp.dot(x_ref[...], w_ref[...], preferred_element_type=jnp.float32)
    g_ref[...] = dinv[:, None] * h
    dinv_ref[...] = dinv


def _stage1(d0, d1, xpad, w1):
    return pl.pallas_call(
        _stage1_body,
        grid=(_GRID,),
        in_specs=[
            pl.BlockSpec((_RB,), lambda i: (i,)),
            pl.BlockSpec((_RB,), lambda i: (i,)),
            pl.BlockSpec((_RB, F), _row_block),
            pl.BlockSpec((F, F), lambda i: (0, 0)),
        ],
        out_specs=[
            pl.BlockSpec((_RB, F), _row_block),
            pl.BlockSpec((_RB,), lambda i: (i,)),
        ],
        out_shape=[
            jax.ShapeDtypeStruct((NPAD, F), jnp.float32),
            jax.ShapeDtypeStruct((NPAD,), jnp.float32),
        ],
    )(d0, d1, xpad, w1)


def _stage_mid_body(p0_ref, p1_ref, g_ref, dinv_ref, b_ref, w_ref, gn_ref):
    dinv = dinv_ref[...]
    t = jnp.tanh(dinv[:, None] * (p0_ref[...] + p1_ref[...] + g_ref[...])
                 + b_ref[...])
    gn_ref[...] = dinv[:, None] * jnp.dot(
        t, w_ref[...], preferred_element_type=jnp.float32)


def _stage_mid(p0, p1, g, dinv, b_row, w_next):
    return pl.pallas_call(
        _stage_mid_body,
        grid=(_GRID,),
        in_specs=[
            pl.BlockSpec((_RB, F), _row_block),
            pl.BlockSpec((_RB, F), _row_block),
            pl.BlockSpec((_RB, F), _row_block),
            pl.BlockSpec((_RB,), lambda i: (i,)),
            pl.BlockSpec((1, F), lambda i: (0, 0)),
            pl.BlockSpec((F, F), lambda i: (0, 0)),
        ],
        out_specs=pl.BlockSpec((_RB, F), _row_block),
        out_shape=jax.ShapeDtypeStruct((NPAD, F), jnp.float32),
    )(p0, p1, g, dinv, b_row, w_next)


_RBF = 2000


def _stage_fin_body(p0_ref, p1_ref, g_ref, dinv_ref, b_ref, o_ref):
    dinv = dinv_ref[...]
    o_ref[...] = (dinv * (p0_ref[...] + p1_ref[...] + g_ref[...])
                  + b_ref[...])


def _stage_fin(p0, p1, g, dinv, b_row):
    return pl.pallas_call(
        _stage_fin_body,
        grid=(N // _RBF,),
        in_specs=[
            pl.BlockSpec((_RBF, F), _row_block),
            pl.BlockSpec((_RBF, F), _row_block),
            pl.BlockSpec((_RBF, F), _row_block),
            pl.BlockSpec((_RBF, 1), lambda i: (i, 0)),
            pl.BlockSpec((1, F), lambda i: (0, 0)),
        ],
        out_specs=pl.BlockSpec((_RBF, F), _row_block),
        out_shape=jax.ShapeDtypeStruct((N, F), jnp.float32),
    )(p0, p1, g, dinv.reshape(NPAD, 1), b_row)


def kernel(x, edge_index, W1, b1, W2, b2, W3, b3):
    src = edge_index[0].astype(jnp.int32)
    dst = edge_index[1].astype(jnp.int32)
    pad = N + (jnp.arange(EPAD - E, dtype=jnp.int32) % (NPAD - N))
    srcp = jnp.concatenate([src, pad]).reshape(NC, NS, CHUNKS, CHUNK)
    dstp = jnp.concatenate([dst, pad]).reshape(NC, NS, CHUNKS, CHUNK)
    xpad = jnp.pad(x, ((0, NPAD - N), (0, 0)))
    b1r, b2r, b3r = (b.reshape(1, F) for b in (b1, b2, b3))

    degp = _deg_kernel(dstp)
    g1, dinv = _stage1(degp[0], degp[1], xpad, W1)
    p = _prop_kernel(g1, srcp, dstp)
    g2 = _stage_mid(p[0], p[1], g1, dinv, b1r, W2)
    p = _prop_kernel(g2, srcp, dstp)
    g3 = _stage_mid(p[0], p[1], g2, dinv, b2r, W3)
    p = _prop_kernel(g3, srcp, dstp)
    return _stage_fin(p[0], p[1], g3, dinv, b3r)
